# baseline (speedup 1.0000x reference)
import math

import jax
import jax.numpy as jnp
import numpy as np
from jax.experimental import pallas as pl
from jax.experimental.pallas import tpu as pltpu

_B, _L, _H, _D = 4, 2048, 16, 64
_U = 5 * int(np.ceil(np.log(_L)))
_TQ = 256
_NEG = -1e30


def _rotl32(x, d):
    d = np.uint32(d)
    return ((x << d) | (x >> (np.uint32(32) - d))).astype(np.uint32)


def _threefry2x32(k0, k1, x0, x1):
    rot = [np.uint32(r) for r in (13, 15, 26, 6, 17, 29, 16, 24)]
    ks0, ks1 = np.uint32(k0), np.uint32(k1)
    ks2 = np.uint32(ks0 ^ ks1 ^ np.uint32(0x1BD11BDA))
    x0 = (x0 + ks0).astype(np.uint32)
    x1 = (x1 + ks1).astype(np.uint32)
    inject = [(ks1, ks2), (ks2, ks0), (ks0, ks1), (ks1, ks2), (ks2, ks0)]
    rounds = [rot[:4], rot[4:], rot[:4], rot[4:], rot[:4]]
    for r in range(5):
        for d in rounds[r]:
            x0 = (x0 + x1).astype(np.uint32)
            x1 = (_rotl32(x1, d) ^ x0).astype(np.uint32)
        a, b = inject[r]
        x0 = (x0 + a).astype(np.uint32)
        x1 = (x1 + b + np.uint32(r + 1)).astype(np.uint32)
    return x0, x1


def _np_random_bits(k0, k1, n):
    b1, b2 = _threefry2x32(
        k0, k1, np.zeros(n, np.uint32), np.arange(n, dtype=np.uint32)
    )
    return b1 ^ b2


def _np_randint(seed, shape, lo, hi):
    o0, o1 = _threefry2x32(
        0, seed, np.zeros(2, np.uint32), np.arange(2, dtype=np.uint32)
    )
    n = int(np.prod(shape))
    hb = _np_random_bits(o0[0], o1[0], n)
    lb = _np_random_bits(o0[1], o1[1], n)
    span = int(hi - lo)
    mult = np.uint32(pow(65536 % span, 2, span))
    val = ((hb % np.uint32(span)) * mult + (lb % np.uint32(span))) % np.uint32(span)
    return (np.int32(lo) + val.astype(np.int32)).reshape(shape)


def _count_matrix_T() -> np.ndarray:
    idx = _np_randint(1234, (_L, _U), 0, _L)
    c = np.zeros((_L, _L), dtype=np.float32)
    np.add.at(c, (np.arange(_L)[:, None], idx), 1.0)
    return np.ascontiguousarray(c.T)


_C_T_HOST = _count_matrix_T()


def _p1_stats(c_ref, q_ref, k_ref, m_ref):
    pk = k_ref[:, :]

    for h in range(2):
        kh = pk[:, h * _D:(h + 1) * _D]

        def qb_body(qi, _, kh=kh, h=h):
            q_blk = q_ref[pl.ds(qi * _TQ, _TQ), h * _D:(h + 1) * _D]
            st = jax.lax.dot_general(
                kh, q_blk, (((1,), (1,)), ((), ())),
                preferred_element_type=jnp.float32,
            )
            c = c_ref[:, pl.ds(qi * _TQ, _TQ)]
            mx = jnp.max(jnp.where(c > 0, st, _NEG), axis=0, keepdims=True)
            sm = jnp.sum(c * st, axis=0, keepdims=True)
            m_ref[pl.ds(h, 1), pl.ds(qi * _TQ, _TQ)] = mx - sm * (1.0 / _L)
            return 0

        jax.lax.fori_loop(0, _L // _TQ, qb_body, 0)


def _p2_topk(m_ref, sel_ref):
    m = m_ref[:, :, :].reshape(_B * _H, _L)
    qiota = jax.lax.broadcasted_iota(jnp.int32, (_B * _H, _L), 1)

    def top_body(i, carry):
        m, sel = carry
        mx = jnp.max(m, axis=1, keepdims=True)
        cand = jnp.where(m == mx, qiota, jnp.int32(_L))
        amin = jnp.min(cand, axis=1, keepdims=True)
        hit = qiota == amin
        sel = jnp.where(hit, i, sel)
        m = jnp.where(hit, _NEG, m)
        return m, sel

    _, sel = jax.lax.fori_loop(
        0, _U, top_body,
        (m, jnp.full((_B * _H, _L), -1, jnp.int32)),
    )
    sel_ref[:, :, :] = sel.reshape(_B * _H // 2, 2, _L)


def _p3_attend(sel_ref, q_ref, k_ref, v_ref, o_ref):
    scale = 1.0 / math.sqrt(_D)
    pq = q_ref[:, :]
    pk = k_ref[:, :]
    pv = v_ref[:, :]
    riota = jax.lax.broadcasted_iota(jnp.int32, (_U, _L), 0)

    for h in range(2):
        qh = pq[:, h * _D:(h + 1) * _D]
        kh = pk[:, h * _D:(h + 1) * _D]
        vh = pv[:, h * _D:(h + 1) * _D]
        sel = sel_ref[pl.ds(h, 1), :]
        onehot = (riota == sel).astype(jnp.float32)

        q_red = jax.lax.dot_general(
            onehot, qh, (((1,), (0,)), ((), ())),
            preferred_element_type=jnp.float32,
        )
        scores = jax.lax.dot_general(
            q_red, kh, (((1,), (1,)), ((), ())),
            preferred_element_type=jnp.float32,
        ) * scale
        smax = jnp.max(scores, axis=1, keepdims=True)
        e = jnp.exp(scores - smax)
        attn = e / jnp.sum(e, axis=1, keepdims=True)
        upd = jax.lax.dot_general(
            attn, vh, (((1,), (0,)), ((), ())),
            preferred_element_type=jnp.float32,
        )

        vmean = jnp.mean(vh, axis=0, keepdims=True)
        ctx = jax.lax.dot_general(
            onehot, upd - vmean, (((0,), (0,)), ((), ())),
            preferred_element_type=jnp.float32,
        ) + vmean
        o_ref[pl.ds(h, 1), :, :] = ctx[None, :, :]


def kernel(queries, keys, values, attn_mask):
    del attn_mask
    B, L, H, D = queries.shape
    BH = B * H
    NP = BH // 2
    qf = queries.reshape(B, L, H * D)
    kf = keys.reshape(B, L, H * D)
    vf = values.reshape(B, L, H * D)
    c_t = jnp.asarray(_C_T_HOST)

    pair_spec = pl.BlockSpec(
        (None, _L, 2 * _D), lambda i: (i // (_H // 2), 0, i % (_H // 2))
    )

    m = pl.pallas_call(
        _p1_stats,
        grid=(NP,),
        in_specs=[
            pl.BlockSpec((_L, _L), lambda i: (0, 0)),
            pair_spec,
            pair_spec,
        ],
        out_specs=pl.BlockSpec((None, 2, _L), lambda i: (i, 0, 0)),
        out_shape=jax.ShapeDtypeStruct((NP, 2, _L), jnp.float32),
        compiler_params=pltpu.CompilerParams(
            dimension_semantics=("arbitrary",),
        ),
    )(c_t, qf, kf)

    sel = pl.pallas_call(
        _p2_topk,
        in_specs=[pl.BlockSpec((NP, 2, _L), lambda: (0, 0, 0))],
        out_specs=pl.BlockSpec((NP, 2, _L), lambda: (0, 0, 0)),
        out_shape=jax.ShapeDtypeStruct((NP, 2, _L), jnp.int32),
    )(m)

    out = pl.pallas_call(
        _p3_attend,
        grid=(NP,),
        in_specs=[
            pl.BlockSpec((None, 2, _L), lambda i: (i, 0, 0)),
            pair_spec,
            pair_spec,
            pair_spec,
        ],
        out_specs=pl.BlockSpec((2, _L, _D), lambda i: (i, 0, 0)),
        out_shape=jax.ShapeDtypeStruct((BH, L, D), jnp.float32),
        compiler_params=pltpu.CompilerParams(
            dimension_semantics=("arbitrary",),
        ),
    )(sel, qf, kf, vf)
    return out.reshape(B, H, L, D)


# speedup vs baseline: 1.1276x; 1.1276x over previous
import functools
import math

import jax
import jax.numpy as jnp
import numpy as np
from jax import lax
from jax.experimental import pallas as pl
from jax.experimental.pallas import tpu as pltpu
from jax.experimental.pallas import tpu_sc as plsc

_B, _L, _H, _D = 4, 2048, 16, 64
_U = 5 * int(np.ceil(np.log(_L)))
_TQ = 256
_CK = 256
_NEG = -1e30


def _rotl32(x, d):
    d = np.uint32(d)
    return ((x << d) | (x >> (np.uint32(32) - d))).astype(np.uint32)


def _threefry2x32(k0, k1, x0, x1):
    rot = [np.uint32(r) for r in (13, 15, 26, 6, 17, 29, 16, 24)]
    ks0, ks1 = np.uint32(k0), np.uint32(k1)
    ks2 = np.uint32(ks0 ^ ks1 ^ np.uint32(0x1BD11BDA))
    x0 = (x0 + ks0).astype(np.uint32)
    x1 = (x1 + ks1).astype(np.uint32)
    inject = [(ks1, ks2), (ks2, ks0), (ks0, ks1), (ks1, ks2), (ks2, ks0)]
    rounds = [rot[:4], rot[4:], rot[:4], rot[4:], rot[:4]]
    for r in range(5):
        for d in rounds[r]:
            x0 = (x0 + x1).astype(np.uint32)
            x1 = (_rotl32(x1, d) ^ x0).astype(np.uint32)
        a, b = inject[r]
        x0 = (x0 + a).astype(np.uint32)
        x1 = (x1 + b + np.uint32(r + 1)).astype(np.uint32)
    return x0, x1


def _np_random_bits(k0, k1, n):
    b1, b2 = _threefry2x32(
        k0, k1, np.zeros(n, np.uint32), np.arange(n, dtype=np.uint32)
    )
    return b1 ^ b2


def _np_randint(seed, shape, lo, hi):
    o0, o1 = _threefry2x32(
        0, seed, np.zeros(2, np.uint32), np.arange(2, dtype=np.uint32)
    )
    n = int(np.prod(shape))
    hb = _np_random_bits(o0[0], o1[0], n)
    lb = _np_random_bits(o0[1], o1[1], n)
    span = int(hi - lo)
    mult = np.uint32(pow(65536 % span, 2, span))
    val = ((hb % np.uint32(span)) * mult + (lb % np.uint32(span))) % np.uint32(span)
    return (np.int32(lo) + val.astype(np.int32)).reshape(shape)


def _count_matrix_T() -> np.ndarray:
    idx = _np_randint(1234, (_L, _U), 0, _L)
    c = np.zeros((_L, _L), dtype=np.float32)
    np.add.at(c, (np.arange(_L)[:, None], idx), 1.0)
    return np.ascontiguousarray(c.T)


_C_T_HOST = _count_matrix_T()
_A_T_HOST = np.where(_C_T_HOST > 0, np.float32(0.0), np.float32(_NEG))


def _p1_stats(c_ref, a_ref, q_ref, k_ref, m_ref):
    def qb_body(qi, _):
        c = c_ref[:, pl.ds(qi * _TQ, _TQ)]
        a = a_ref[:, pl.ds(qi * _TQ, _TQ)]
        for h in range(2):
            q_blk = q_ref[pl.ds(qi * _TQ, _TQ), h * _D:(h + 1) * _D]
            kh = k_ref[:, h * _D:(h + 1) * _D]
            st = jax.lax.dot_general(
                kh, q_blk, (((1,), (1,)), ((), ())),
                preferred_element_type=jnp.float32,
            )
            stm = st + a
            mx = jnp.max(stm, axis=0, keepdims=True)
            sm = jnp.sum(c * stm, axis=0, keepdims=True)
            m_ref[pl.ds(h, 1), pl.ds(qi * _TQ, _TQ)] = mx - sm * (1.0 / _L)
        return 0

    jax.lax.fori_loop(0, _L // _TQ, qb_body, 0)


def _p2_topk(m_ref, sel_ref, idx_ref):
    m = m_ref[:, :, :].reshape(_B * _H, _L)
    qiota = jax.lax.broadcasted_iota(jnp.int32, (_B * _H, _L), 1)

    def top_body(i, carry):
        m, sel = carry
        mx = jnp.max(m, axis=1, keepdims=True)
        cand = jnp.where(m == mx, qiota, jnp.int32(_L))
        amin = jnp.min(cand, axis=1, keepdims=True)
        idx_ref[pl.ds(i, 1), :] = amin.reshape(1, _B * _H)
        hit = qiota == amin
        sel = jnp.where(hit, i, sel)
        m = jnp.where(hit, _NEG, m)
        return m, sel

    _, sel = jax.lax.fori_loop(
        0, _U, top_body,
        (m, jnp.full((_B * _H, _L), -1, jnp.int32)),
    )
    sel_ref[:, :, :] = sel.reshape(_B * _H // 2, 2, _L)


def _sc_gather_qrows():
    info = plsc.get_sparse_core_info()
    nc, ns = info.num_cores, info.num_subcores
    nrows = _B * _H * _U
    b_per_w = nrows // (nc * ns)
    mesh = plsc.VectorSubcoreMesh(core_axis_name="c", subcore_axis_name="s")

    @functools.partial(
        pl.kernel, mesh=mesh,
        out_type=jax.ShapeDtypeStruct((nrows, 2 * _D), jnp.float32),
        scratch_types=[
            pltpu.VMEM((b_per_w,), jnp.int32),
            pltpu.VMEM((b_per_w, 2 * _D), jnp.float32),
            pltpu.SemaphoreType.DMA,
        ],
    )
    def gather(table_hbm, idx_hbm, out_hbm, idx_v, rows_v, sem):
        wid = lax.axis_index("s") * nc + lax.axis_index("c")
        base = wid * b_per_w
        pltpu.sync_copy(idx_hbm.at[pl.ds(base, b_per_w)], idx_v)
        pltpu.async_copy(table_hbm.at[idx_v], rows_v, sem).wait()
        pltpu.sync_copy(rows_v, out_hbm.at[pl.ds(base, b_per_w)])

    return gather


def _p3_attend(sel_ref, qred_ref, k_ref, v_ref, o_ref):
    scale = 1.0 / math.sqrt(_D)
    pk = k_ref[:, :]
    pv = v_ref[:, :]
    riota = jax.lax.broadcasted_iota(jnp.int32, (_U, _L), 0)

    for h in range(2):
        kh = pk[:, h * _D:(h + 1) * _D]
        vh = pv[:, h * _D:(h + 1) * _D]
        sel = sel_ref[pl.ds(h, 1), :]
        onehot = (riota == sel).astype(jnp.float32)

        q_red = qred_ref[h, :, h * _D:(h + 1) * _D]
        scores = jax.lax.dot_general(
            q_red, kh, (((1,), (1,)), ((), ())),
            preferred_element_type=jnp.float32,
        ) * scale
        smax = jnp.max(scores, axis=1, keepdims=True)
        e = jnp.exp(scores - smax)
        attn = e / jnp.sum(e, axis=1, keepdims=True)
        upd = jax.lax.dot_general(
            attn, vh, (((1,), (0,)), ((), ())),
            preferred_element_type=jnp.float32,
        )

        vmean = jnp.mean(vh, axis=0, keepdims=True)
        ctx = jax.lax.dot_general(
            onehot, upd - vmean, (((0,), (0,)), ((), ())),
            preferred_element_type=jnp.float32,
        ) + vmean
        o_ref[pl.ds(h, 1), :, :] = ctx[None, :, :]


def kernel(queries, keys, values, attn_mask):
    del attn_mask
    B, L, H, D = queries.shape
    BH = B * H
    NP = BH // 2
    qf = queries.reshape(B, L, H * D)
    kf = keys.reshape(B, L, H * D)
    vf = values.reshape(B, L, H * D)
    c_t = jnp.asarray(_C_T_HOST)
    a_t = jnp.asarray(_A_T_HOST)

    pair_spec = pl.BlockSpec(
        (None, _L, 2 * _D), lambda i: (i // (_H // 2), 0, i % (_H // 2))
    )

    m = pl.pallas_call(
        _p1_stats,
        grid=(NP,),
        in_specs=[
            pl.BlockSpec((_L, _L), lambda i: (0, 0)),
            pl.BlockSpec((_L, _L), lambda i: (0, 0)),
            pair_spec,
            pair_spec,
        ],
        out_specs=pl.BlockSpec((None, 2, _L), lambda i: (i, 0, 0)),
        out_shape=jax.ShapeDtypeStruct((NP, 2, _L), jnp.float32),
        compiler_params=pltpu.CompilerParams(
            dimension_semantics=("arbitrary",),
        ),
    )(c_t, a_t, qf, kf)

    sel, idx = pl.pallas_call(
        _p2_topk,
        in_specs=[pl.BlockSpec((NP, 2, _L), lambda: (0, 0, 0))],
        out_specs=(
            pl.BlockSpec((NP, 2, _L), lambda: (0, 0, 0)),
            pl.BlockSpec((_U, BH), lambda: (0, 0)),
        ),
        out_shape=(
            jax.ShapeDtypeStruct((NP, 2, _L), jnp.int32),
            jax.ShapeDtypeStruct((_U, BH), jnp.int32),
        ),
    )(m)

    heads = jnp.arange(BH, dtype=jnp.int32)
    hb, hh = heads // H, heads % H
    idx_flat = (
        (hb[:, None] * L + idx.T) * (H // 2) + hh[:, None] // 2
    ).reshape(BH * _U)

    qred = _sc_gather_qrows()(qf.reshape(B * L * (H // 2), 2 * D), idx_flat)
    qred = qred.reshape(NP, 2, _U, 2 * _D)

    out = pl.pallas_call(
        _p3_attend,
        grid=(NP,),
        in_specs=[
            pl.BlockSpec((None, 2, _L), lambda i: (i, 0, 0)),
            pl.BlockSpec((None, 2, _U, 2 * _D), lambda i: (i, 0, 0, 0)),
            pair_spec,
            pair_spec,
        ],
        out_specs=pl.BlockSpec((2, _L, _D), lambda i: (i, 0, 0)),
        out_shape=jax.ShapeDtypeStruct((BH, L, D), jnp.float32),
        compiler_params=pltpu.CompilerParams(
            dimension_semantics=("arbitrary",),
        ),
    )(sel, qred, kf, vf)
    return out.reshape(B, H, L, D)
